# baseline (speedup 1.0000x reference)
import functools
import math

import jax
import jax.numpy as jnp
from jax import lax
from jax.experimental import pallas as pl
from jax.experimental.pallas import tpu as pltpu
from jax.experimental.pallas import tpu_sc as plsc

_NC = 2
_NS = 16
_NW = _NC * _NS
_L = 16

_SUP = 4096
_GB = 128
_DEPTH = 4
_SIGMA = 0.003


@functools.lru_cache(maxsize=None)
def _make_relayout_kernel(H, W):
    HPW = H // _NW
    mesh = plsc.VectorSubcoreMesh(core_axis_name="c", subcore_axis_name="s")

    @functools.partial(
        pl.kernel,
        out_type=jax.ShapeDtypeStruct((H * W, 32), jnp.float32),
        mesh=mesh,
        compiler_params=pltpu.CompilerParams(
            needs_layout_passes=False, use_tc_tiling_on_sc=False),
        scratch_types=[
            [pltpu.VMEM((2, 4, 4, 4, 128), jnp.float32) for _ in range(3)],
            [pltpu.VMEM((W // 2, 33), jnp.float32) for _ in range(3)],
            [pltpu.SemaphoreType.DMA for _ in range(3)],
            [pltpu.SemaphoreType.DMA for _ in range(3)],
        ],
    )
    def relayout(z_h, out_h, zin, ost, zsem, osem):
        wid = lax.axis_index("c") * _NS + lax.axis_index("s")
        h0 = wid * HPW
        nchunk = HPW * 2
        riota = lax.iota(jnp.int32, _L)

        def zin_start(cc, par):
            h = h0 + lax.div(cc, 2)
            th = lax.rem(cc, 2)
            pltpu.make_async_copy(
                z_h.at[h, :, :, pl.ds(th * 4, 4)], zin[par], zsem[par]).start()

        def out_descr(cc, par):
            h = h0 + lax.div(cc, 2)
            th = lax.rem(cc, 2)
            return pltpu.make_async_copy(
                ost[par].at[:, pl.ds(0, 32)],
                out_h.at[pl.ds(h * W + th * (W // 2), W // 2)], osem[par])

        zin_start(0, 0)
        zin_start(1, 1)
        zin_start(2, 2)

        def cc2_body(cc2, carry):
            for par in range(3):
                cc = cc2 * 3 + par
                pltpu.make_async_copy(
                    z_h.at[h0, :, :, pl.ds(0, 4)], zin[par], zsem[par]).wait()

                @pl.when(cc >= 3)
                def _():
                    out_descr(cc, par).wait()

                def g_body(g, c3):
                    t_rel = lax.div(g, 8)
                    l0 = lax.rem(g, 8) * _L
                    rows16 = t_rel * 128 + l0 + riota
                    for kk in range(32):
                        ch, ii, jj = kk >> 4, (kk >> 2) & 3, kk & 3
                        val = zin[par][ch, ii, t_rel, jj, pl.ds(l0, _L)]
                        plsc.store_scatter(
                            ost[par], [rows16, jnp.full((_L,), kk, jnp.int32)],
                            val)
                    return c3
                lax.fori_loop(0, 32, g_body, 0)

                out_descr(cc, par).start()

                @pl.when(cc + 3 < nchunk)
                def _():
                    zin_start(cc + 3, par)
            return carry
        lax.fori_loop(0, nchunk // 3, cc2_body, 0)
        cc_last = (nchunk // 3) * 3
        par_last = 0
        pltpu.make_async_copy(
            z_h.at[h0, :, :, pl.ds(0, 4)], zin[par_last], zsem[par_last]).wait()
        out_descr(cc_last, par_last).wait()

        def g_tail(g, c3):
            t_rel = lax.div(g, 8)
            l0 = lax.rem(g, 8) * _L
            rows16 = t_rel * 128 + l0 + riota
            for kk in range(32):
                ch, ii, jj = kk >> 4, (kk >> 2) & 3, kk & 3
                val = zin[par_last][ch, ii, t_rel, jj, pl.ds(l0, _L)]
                plsc.store_scatter(
                    ost[par_last], [rows16, jnp.full((_L,), kk, jnp.int32)], val)
            return c3
        lax.fori_loop(0, 32, g_tail, 0)
        out_descr(cc_last, par_last).start()
        out_descr(cc_last, par_last).wait()
        for par in range(1, 3):
            out_descr(cc_last - 3 + par, par).wait()

    return relayout

@functools.lru_cache(maxsize=None)
def _make_sc_kernel(B, H, W):
    D = 32
    bpw = B // _NW
    nsup = bpw // _SUP
    nblk = _SUP // _GB
    ngrp = _GB // _L
    assert B % _NW == 0 and bpw % _SUP == 0 and _SUP % _GB == 0
    assert nblk % _DEPTH == 0
    assert (H & (H - 1)) == 0 and (W & (W - 1)) == 0

    Hf, Wf = float(H), float(W)
    knorm = 1.0 / (2.0 * math.pi * _SIGMA)

    mesh = plsc.VectorSubcoreMesh(core_axis_name="c", subcore_axis_name="s")

    @functools.partial(
        pl.kernel,
        out_type=jax.ShapeDtypeStruct((B,), jnp.float32),
        mesh=mesh,
        compiler_params=pltpu.CompilerParams(
            needs_layout_passes=False, use_tc_tiling_on_sc=False),
        scratch_types=[
            pltpu.VMEM((_SUP // 128, 2, 128), jnp.float32),
            pltpu.VMEM((_SUP // 128, 2, 128), jnp.float32),
            pltpu.VMEM((_SUP,), jnp.float32),
            pltpu.VMEM((_SUP,), jnp.float32),
            pltpu.VMEM((_SUP,), jnp.int32),
            [pltpu.VMEM((_GB, D), jnp.float32) for _ in range(_DEPTH)],
            pltpu.VMEM((_GB // _L, 17 * D), jnp.float32),
            pltpu.VMEM((_SUP,), jnp.float32),
            [pltpu.SemaphoreType.DMA for _ in range(_DEPTH)],
        ],
    )
    def ndf_kernel(u_h, s_h, tab_h, out_h,
                   uv_v, sv_v, uf_v, vf_v, idx_v,
                   rows, pad_v, o_v, sems):
        wid = lax.axis_index("c") * _NS + lax.axis_index("s")
        wbase = wid * bpw
        iota = lax.iota(jnp.int32, _L)
        iota17 = iota * 17

        def gather_start(j, rbuf, sem):
            pltpu.make_async_copy(
                tab_h.at[idx_v.at[pl.ds(j * _GB, _GB)]], rbuf, sem).start()

        def gather_wait(j, rbuf, sem):
            pltpu.make_async_copy(
                tab_h.at[idx_v.at[pl.ds(j * _GB, _GB)]], rbuf, sem).wait()

        def sup_body(sc, carry):
            base = wbase + sc * _SUP
            pltpu.sync_copy(u_h.at[pl.ds(base // 128, _SUP // 128)], uv_v)
            pltpu.sync_copy(s_h.at[pl.ds(base // 128, _SUP // 128)], sv_v)

            @plsc.parallel_loop(0, _SUP // _L, unroll=4)
            def idx_body(g):
                o = g * _L
                t = lax.div(o, 128)
                lo = lax.rem(o, 128)
                a0 = uv_v[t, 0, pl.ds(lo, _L)]
                a1 = uv_v[t, 1, pl.ds(lo, _L)]
                vv = (a0 * 0.5 + 0.5) * Hf
                uu = (a1 * 0.5 + 0.5) * Wf
                vi = vv.astype(jnp.int32)
                ui = uu.astype(jnp.int32)
                vf_v[pl.ds(o, _L)] = vv - vi.astype(jnp.float32)
                uf_v[pl.ds(o, _L)] = uu - ui.astype(jnp.float32)
                idx_v[pl.ds(o, _L)] = (ui & (H - 1)) * W + (vi & (W - 1))

            for d in range(_DEPTH):
                gather_start(d, rows[d], sems[d])

            def compute_block(j, rbuf):
                def grp(k, _carry):
                    o = j * _GB + k * _L
                    for l in range(_L):
                        pt = k * _L + l
                        a = rbuf[pt, pl.ds(0, _L)]
                        b = rbuf[pt, pl.ds(_L, _L)]
                        plsc.store_scatter(pad_v, [jnp.full((_L,), k, jnp.int32), iota17 + l], a)
                        plsc.store_scatter(pad_v, [jnp.full((_L,), k, jnp.int32), iota17 + (l + 17 * _L)], b)
                    cs = [pad_v[k, pl.ds(kk * 17, _L)] for kk in range(D)]
                    uf = uf_v[pl.ds(o, _L)]
                    vf = vf_v[pl.ds(o, _L)]

                    def h4(c0, c1, c2, c3, t):
                        return ((c3 * t + c2) * t + c1) * t + c0

                    ns = []
                    for ch in range(2):
                        rs = [h4(*cs[ch * 16 + i * 4: ch * 16 + i * 4 + 4], vf)
                              for i in range(4)]
                        ns.append(h4(rs[0], rs[1], rs[2], rs[3], uf))
                    st = lax.div(o, 128)
                    slo = lax.rem(o, 128)
                    d0 = (ns[0] - sv_v[st, 0, pl.ds(slo, _L)]) / _SIGMA
                    d1 = (ns[1] - sv_v[st, 1, pl.ds(slo, _L)]) / _SIGMA
                    o_v[pl.ds(o, _L)] = knorm * jnp.exp(-0.5 * (d0 * d0 + d1 * d1))
                    return _carry
                lax.fori_loop(0, ngrp, grp, 0)

            def blkn(jj, c):
                j0 = jj * _DEPTH
                for par in range(_DEPTH):
                    j = j0 + par
                    gather_wait(j, rows[par], sems[par])
                    compute_block(j, rows[par])

                    @pl.when(j + _DEPTH < nblk)
                    def _():
                        gather_start(j + _DEPTH, rows[par], sems[par])
                return c
            lax.fori_loop(0, nblk // _DEPTH, blkn, 0)

            pltpu.sync_copy(o_v, out_h.at[pl.ds(base, _SUP)])
            return carry
        lax.fori_loop(0, nsup, sup_body, 0)

    return ndf_kernel


def kernel(u, s, normal_coeff):
    H, W, C = normal_coeff.shape[0], normal_coeff.shape[1], normal_coeff.shape[2]
    B = u.shape[0]
    assert C == 2
    Z = (normal_coeff.transpose(0, 2, 3, 4, 1)
         .reshape(H, C, 4, 4, W // 128, 128)
         .transpose(0, 1, 2, 4, 3, 5))
    table = _make_relayout_kernel(H, W)(Z)
    uV = u.reshape(B // 128, 128, 2).transpose(0, 2, 1)
    sV = s.reshape(B // 128, 128, 2).transpose(0, 2, 1)
    f = _make_sc_kernel(B, H, W)
    return f(uV, sV, table)


# speedup vs baseline: 1.0022x; 1.0022x over previous
import functools
import math

import jax
import jax.numpy as jnp
from jax import lax
from jax.experimental import pallas as pl
from jax.experimental.pallas import tpu as pltpu
from jax.experimental.pallas import tpu_sc as plsc

_NC = 2
_NS = 16
_NW = _NC * _NS
_L = 16

_SUP = 4096
_GB = 128
_DEPTH = 4
_SIGMA = 0.003


@functools.lru_cache(maxsize=None)
def _make_relayout_kernel(H, W):
    HPW = H // _NW
    mesh = plsc.VectorSubcoreMesh(core_axis_name="c", subcore_axis_name="s")

    @functools.partial(
        pl.kernel,
        out_type=jax.ShapeDtypeStruct((H * W, 32), jnp.float32),
        mesh=mesh,
        compiler_params=pltpu.CompilerParams(
            needs_layout_passes=False, use_tc_tiling_on_sc=False),
        scratch_types=[
            [pltpu.VMEM((2, 4, 4, 4, 128), jnp.float32) for _ in range(2)],
            [pltpu.VMEM((W // 2, 33), jnp.float32) for _ in range(2)],
            [pltpu.SemaphoreType.DMA for _ in range(2)],
            [pltpu.SemaphoreType.DMA for _ in range(2)],
        ],
    )
    def relayout(z_h, out_h, zin, ost, zsem, osem):
        wid = lax.axis_index("c") * _NS + lax.axis_index("s")
        h0 = wid * HPW
        nchunk = HPW * 2
        riota = lax.iota(jnp.int32, _L)

        def zin_start(cc, par):
            h = h0 + lax.div(cc, 2)
            th = lax.rem(cc, 2)
            pltpu.make_async_copy(
                z_h.at[h, :, :, pl.ds(th * 4, 4)], zin[par], zsem[par]).start()

        def out_descr(cc, par):
            h = h0 + lax.div(cc, 2)
            th = lax.rem(cc, 2)
            return pltpu.make_async_copy(
                ost[par].at[:, pl.ds(0, 32)],
                out_h.at[pl.ds(h * W + th * (W // 2), W // 2)], osem[par])

        zin_start(0, 0)
        zin_start(1, 1)

        def cc2_body(cc2, carry):
            for par in range(2):
                cc = cc2 * 2 + par
                pltpu.make_async_copy(
                    z_h.at[h0, :, :, pl.ds(0, 4)], zin[par], zsem[par]).wait()

                @pl.when(cc >= 2)
                def _():
                    out_descr(cc, par).wait()

                def g_body(g, c3):
                    t_rel = lax.div(g, 8)
                    l0 = lax.rem(g, 8) * _L
                    rows16 = t_rel * 128 + l0 + riota
                    for kk in range(32):
                        ch, ii, jj = kk >> 4, (kk >> 2) & 3, kk & 3
                        val = zin[par][ch, ii, t_rel, jj, pl.ds(l0, _L)]
                        plsc.store_scatter(
                            ost[par], [rows16, jnp.full((_L,), kk, jnp.int32)],
                            val)
                    return c3
                lax.fori_loop(0, 32, g_body, 0)

                out_descr(cc, par).start()

                @pl.when(cc + 2 < nchunk)
                def _():
                    zin_start(cc + 2, par)
            return carry
        lax.fori_loop(0, nchunk // 2, cc2_body, 0)
        for par in range(2):
            out_descr(nchunk - 2 + par, par).wait()

    return relayout

@functools.lru_cache(maxsize=None)
def _make_sc_kernel(B, H, W):
    D = 32
    bpw = B // _NW
    nsup = bpw // _SUP
    nblk = _SUP // _GB
    ngrp = _GB // _L
    assert B % _NW == 0 and bpw % _SUP == 0 and _SUP % _GB == 0
    assert nblk % _DEPTH == 0
    assert (H & (H - 1)) == 0 and (W & (W - 1)) == 0

    Hf, Wf = float(H), float(W)
    knorm = 1.0 / (2.0 * math.pi * _SIGMA)

    mesh = plsc.VectorSubcoreMesh(core_axis_name="c", subcore_axis_name="s")

    @functools.partial(
        pl.kernel,
        out_type=jax.ShapeDtypeStruct((B,), jnp.float32),
        mesh=mesh,
        compiler_params=pltpu.CompilerParams(
            needs_layout_passes=False, use_tc_tiling_on_sc=False),
        scratch_types=[
            pltpu.VMEM((_SUP // 128, 2, 128), jnp.float32),
            pltpu.VMEM((_SUP // 128, 2, 128), jnp.float32),
            pltpu.VMEM((_SUP,), jnp.float32),
            pltpu.VMEM((_SUP,), jnp.float32),
            pltpu.VMEM((_SUP,), jnp.int32),
            [pltpu.VMEM((_GB, D), jnp.float32) for _ in range(_DEPTH)],
            pltpu.VMEM((_GB // _L, 17 * D), jnp.float32),
            pltpu.VMEM((_SUP,), jnp.float32),
            [pltpu.SemaphoreType.DMA for _ in range(_DEPTH)],
        ],
    )
    def ndf_kernel(u_h, s_h, tab_h, out_h,
                   uv_v, sv_v, uf_v, vf_v, idx_v,
                   rows, pad_v, o_v, sems):
        wid = lax.axis_index("c") * _NS + lax.axis_index("s")
        wbase = wid * bpw
        iota = lax.iota(jnp.int32, _L)
        iota17 = iota * 17

        def gather_start(j, rbuf, sem):
            pltpu.make_async_copy(
                tab_h.at[idx_v.at[pl.ds(j * _GB, _GB)]], rbuf, sem).start()

        def gather_wait(j, rbuf, sem):
            pltpu.make_async_copy(
                tab_h.at[idx_v.at[pl.ds(j * _GB, _GB)]], rbuf, sem).wait()

        def sup_body(sc, carry):
            base = wbase + sc * _SUP
            pltpu.sync_copy(u_h.at[pl.ds(base // 128, _SUP // 128)], uv_v)
            pltpu.sync_copy(s_h.at[pl.ds(base // 128, _SUP // 128)], sv_v)

            @plsc.parallel_loop(0, _SUP // _L, unroll=4)
            def idx_body(g):
                o = g * _L
                t = lax.div(o, 128)
                lo = lax.rem(o, 128)
                a0 = uv_v[t, 0, pl.ds(lo, _L)]
                a1 = uv_v[t, 1, pl.ds(lo, _L)]
                vv = (a0 * 0.5 + 0.5) * Hf
                uu = (a1 * 0.5 + 0.5) * Wf
                vi = vv.astype(jnp.int32)
                ui = uu.astype(jnp.int32)
                vf_v[pl.ds(o, _L)] = vv - vi.astype(jnp.float32)
                uf_v[pl.ds(o, _L)] = uu - ui.astype(jnp.float32)
                idx_v[pl.ds(o, _L)] = (ui & (H - 1)) * W + (vi & (W - 1))

            for d in range(_DEPTH):
                gather_start(d, rows[d], sems[d])

            def compute_block(j, rbuf):
                def grp(k, _carry):
                    o = j * _GB + k * _L
                    for l in range(_L):
                        pt = k * _L + l
                        a = rbuf[pt, pl.ds(0, _L)]
                        b = rbuf[pt, pl.ds(_L, _L)]
                        plsc.store_scatter(pad_v, [jnp.full((_L,), k, jnp.int32), iota17 + l], a)
                        plsc.store_scatter(pad_v, [jnp.full((_L,), k, jnp.int32), iota17 + (l + 17 * _L)], b)
                    cs = [pad_v[k, pl.ds(kk * 17, _L)] for kk in range(D)]
                    uf = uf_v[pl.ds(o, _L)]
                    vf = vf_v[pl.ds(o, _L)]

                    def h4(c0, c1, c2, c3, t):
                        return ((c3 * t + c2) * t + c1) * t + c0

                    ns = []
                    for ch in range(2):
                        rs = [h4(*cs[ch * 16 + i * 4: ch * 16 + i * 4 + 4], vf)
                              for i in range(4)]
                        ns.append(h4(rs[0], rs[1], rs[2], rs[3], uf))
                    st = lax.div(o, 128)
                    slo = lax.rem(o, 128)
                    d0 = (ns[0] - sv_v[st, 0, pl.ds(slo, _L)]) / _SIGMA
                    d1 = (ns[1] - sv_v[st, 1, pl.ds(slo, _L)]) / _SIGMA
                    o_v[pl.ds(o, _L)] = knorm * jnp.exp(-0.5 * (d0 * d0 + d1 * d1))
                    return _carry
                lax.fori_loop(0, ngrp, grp, 0)

            def blkn(jj, c):
                j0 = jj * _DEPTH
                for par in range(_DEPTH):
                    j = j0 + par
                    gather_wait(j, rows[par], sems[par])
                    compute_block(j, rows[par])

                    @pl.when(j + _DEPTH < nblk)
                    def _():
                        gather_start(j + _DEPTH, rows[par], sems[par])
                return c
            lax.fori_loop(0, nblk // _DEPTH, blkn, 0)

            pltpu.sync_copy(o_v, out_h.at[pl.ds(base, _SUP)])
            return carry
        lax.fori_loop(0, nsup, sup_body, 0)

    return ndf_kernel


def kernel(u, s, normal_coeff):
    H, W, C = normal_coeff.shape[0], normal_coeff.shape[1], normal_coeff.shape[2]
    B = u.shape[0]
    assert C == 2
    Z = (normal_coeff.transpose(0, 2, 3, 4, 1)
         .reshape(H, C, 4, 4, W // 128, 128)
         .transpose(0, 1, 2, 4, 3, 5))
    table = _make_relayout_kernel(H, W)(Z)
    uV = u.reshape(B // 128, 128, 2).transpose(0, 2, 1)
    sV = s.reshape(B // 128, 128, 2).transpose(0, 2, 1)
    f = _make_sc_kernel(B, H, W)
    return f(uV, sV, table)
